# baseline (speedup 1.0000x reference)
import functools

import jax
import jax.numpy as jnp
from jax import lax
from jax.experimental import pallas as pl
from jax.experimental.pallas import tpu as pltpu
from jax.experimental.pallas import tpu_sc as plsc

N = 10000
E = 320000
B = 64
IN = 128
HID = 32

NP = 10240
NW = 32
CHUNK = 128
ET = E + N
KCH = 2 * (-(-ET // (NW * CHUNK * 2)))
EP = NW * KCH * CHUNK
RPT = NP // 16


def _mm2_body(x_ref, wl_ref, blr, wr_ref, brr, xl_ref, xr_ref):
    xv = x_ref[...]
    xl_ref[...] = jnp.dot(xv, wl_ref[...], preferred_element_type=jnp.float32) + blr[...]
    xr_ref[...] = jnp.dot(xv, wr_ref[...], preferred_element_type=jnp.float32) + brr[...]


def _dual_matmul(xp, Wl, bl, Wr, br, dout):
    return pl.pallas_call(
        _mm2_body,
        out_shape=[jax.ShapeDtypeStruct((NP, dout), jnp.float32)] * 2,
    )(xp, Wl, bl.reshape(1, dout), Wr, br.reshape(1, dout))


def _sc_edge_sweep(H, C, xl, xr, src3, dst3, att2d, zrow):
    DIN = H * C
    WOUT = DIN + 16
    NV = DIN // 16
    VPH = C // 16

    rot = (jnp.arange(16)[None, :] + jnp.arange(C)[:, None]) % C
    attrot = att2d[:, rot].reshape(DIN, 16)

    mesh = plsc.VectorSubcoreMesh(core_axis_name="c", subcore_axis_name="s")

    def body(xl_hbm, xr_hbm, src_hbm, dst_hbm, att_hbm, zero_hbm, out_hbm,
             sidx, didx, didxs, xlr, xrr, wbuf, attv, accs,
             isem, gsem, ssem):
        c = lax.axis_index("c")
        s = lax.axis_index("s")
        w = c * 16 + s

        pltpu.sync_copy(zero_hbm.at[pl.ds(s * RPT, RPT)],
                        accs.at[pl.ds(s * RPT, RPT)])
        pltpu.sync_copy(att_hbm, attv)
        plsc.subcore_barrier()

        io = lax.iota(jnp.int32, 16)
        cmask = jnp.full((16,), C - 1, jnp.int32)
        PF = 10

        def compute_chunk(a):
            @plsc.parallel_loop(0, CHUNK // 16)
            def group(g):
                rows = g * 16 + io

                def cvec_of(h, t):
                    return ((io + t) & cmask) + (h * C)

                def loads_at(h, t):
                    cv = cvec_of(h, t)
                    return (cv,
                            plsc.load_gather(xlr.at[a], [rows, cv]),
                            plsc.load_gather(xrr.at[a], [rows, cv]))

                hts = [(h, t) for h in range(H) for t in range(C)]
                pend = [loads_at(*hts[i]) for i in range(PF)]
                accs_h = [jnp.zeros((16,), jnp.float32) for _ in range(H)]
                for i, (h, t) in enumerate(hts):
                    if i + PF < len(hts):
                        pend.append(loads_at(*hts[i + PF]))
                    _, xa, xb = pend.pop(0)
                    m = xa + xb
                    lk = jnp.maximum(m, 0.2 * m)
                    accs_h[h] = accs_h[h] + lk * attv[h * C + t]
                exv_h = [jnp.exp(acc) for acc in accs_h]

                def oloads_at(h, t):
                    cv = cvec_of(h, t)
                    return (cv, plsc.load_gather(xlr.at[a], [rows, cv]))

                pend = [oloads_at(*hts[i]) for i in range(PF)]
                for i, (h, t) in enumerate(hts):
                    if i + PF < len(hts):
                        pend.append(oloads_at(*hts[i + PF]))
                    cv, xa = pend.pop(0)
                    plsc.store_scatter(wbuf.at[a], [rows, cv], exv_h[h] * xa)
                for h in range(H):
                    dcol = io * 0 + (DIN + h)
                    plsc.store_scatter(wbuf.at[a], [rows, dcol], exv_h[h])

        def wait_gathers(a):
            pltpu.make_async_copy(xl_hbm.at[sidx.at[a]], xlr.at[a], gsem.at[a]).wait()
            pltpu.make_async_copy(xr_hbm.at[didx.at[a]], xrr.at[a], gsem.at[a]).wait()

        def fire_gathers(a):
            pltpu.async_copy(xl_hbm.at[sidx.at[a]], xlr.at[a], gsem.at[a])
            pltpu.async_copy(xr_hbm.at[didx.at[a]], xrr.at[a], gsem.at[a])

        def fire_idx(j, a):
            pltpu.async_copy(src_hbm.at[w, j], sidx.at[a], isem.at[a])
            pltpu.async_copy(dst_hbm.at[w, j], didx.at[a], isem.at[a])

        def wait_idx(a):
            pltpu.make_async_copy(src_hbm.at[w, 0], sidx.at[a], isem.at[a]).wait()
            pltpu.make_async_copy(dst_hbm.at[w, 0], didx.at[a], isem.at[a]).wait()

        def fire_scatter(a):
            pltpu.async_copy(wbuf.at[a], accs.at[didxs.at[a]], ssem.at[a], add=True)

        def wait_scatter(a):
            pltpu.make_async_copy(wbuf.at[a], accs.at[didxs.at[a]], ssem.at[a]).wait()

        pltpu.sync_copy(src_hbm.at[w, 0], sidx.at[0])
        pltpu.sync_copy(dst_hbm.at[w, 0], didx.at[0])
        fire_gathers(0)
        fire_idx(1, 1)

        def halfstep(j, a, b):
            wait_gathers(a)

            @pl.when(j >= 2)
            def _():
                wait_scatter(a)

            for k in range(CHUNK // 16):
                didxs[a, pl.ds(k * 16, 16)] = didx[a, pl.ds(k * 16, 16)]

            @pl.when(j + 1 < KCH)
            def _():
                wait_idx(b)
                fire_gathers(b)

            @pl.when(j + 2 < KCH)
            def _():
                fire_idx(j + 2, a)

            compute_chunk(a)
            fire_scatter(a)

        def pipe(jj, carry):
            halfstep(2 * jj, 0, 1)
            halfstep(2 * jj + 1, 1, 0)
            return carry

        lax.fori_loop(0, KCH // 2, pipe, 0)
        wait_scatter(0)
        wait_scatter(1)
        plsc.subcore_barrier()
        pltpu.sync_copy(accs.at[pl.ds(s * RPT, RPT)],
                        out_hbm.at[c, pl.ds(s * RPT, RPT)])

    f = pl.kernel(
        body,
        out_type=jax.ShapeDtypeStruct((2, NP, WOUT), jnp.float32),
        mesh=mesh,
        compiler_params=pltpu.CompilerParams(
            needs_layout_passes=False, use_tc_tiling_on_sc=False),
        scratch_types=[
            pltpu.VMEM((2, CHUNK), jnp.int32),
            pltpu.VMEM((2, CHUNK), jnp.int32),
            pltpu.VMEM((2, CHUNK), jnp.int32),
            pltpu.VMEM((2, CHUNK, DIN), jnp.float32),
            pltpu.VMEM((2, CHUNK, DIN), jnp.float32),
            pltpu.VMEM((2, CHUNK, WOUT), jnp.float32),
            pltpu.VMEM((DIN, 16), jnp.float32),
            pltpu.VMEM_SHARED((NP, WOUT), jnp.float32),
            pltpu.SemaphoreType.DMA((2,)),
            pltpu.SemaphoreType.DMA((2,)),
            pltpu.SemaphoreType.DMA((2,)),
        ],
    )
    return f(xl, xr, src3, dst3, attrot, zrow)


def _mid_body(acc_ref, bias_ref, wl_ref, blr, wr_ref, brr, xl_ref, xr_ref):
    a = acc_ref[0] + acc_ref[1]
    num = a[:, :64]
    den = jnp.concatenate(
        [jnp.broadcast_to(a[:, 64:65], (NP, 32)),
         jnp.broadcast_to(a[:, 65:66], (NP, 32))], axis=1)
    h = jnp.maximum(num / den + bias_ref[...], 0.0)
    xl_ref[...] = jnp.dot(h, wl_ref[...], preferred_element_type=jnp.float32) + blr[...]
    xr_ref[...] = jnp.dot(h, wr_ref[...], preferred_element_type=jnp.float32) + brr[...]


def _final_body(acc_ref, bias_ref, bcol_ref, brow_ref, wg1_ref, bg1_ref,
                wg2_ref, bg2_ref, w3_ref, b3_ref, w4_ref, b4_ref, out_ref):
    a = acc_ref[0] + acc_ref[1]
    h2 = jnp.maximum(a[:, :32] / a[:, 32:33] + bias_ref[...], 0.0)
    rows = lax.broadcasted_iota(jnp.int32, (NP, 1), 0)
    h2 = jnp.where(rows < N, h2, 0.0)

    g1 = jnp.maximum(jnp.dot(h2, wg1_ref[...], preferred_element_type=jnp.float32) + bg1_ref[...], 0.0)
    gate = jnp.dot(g1, wg2_ref[...], preferred_element_type=jnp.float32) + bg2_ref[...]

    bcol = bcol_ref[...]
    onehot = bcol == lax.broadcasted_iota(jnp.int32, (NP, B), 1)
    onehot_t = brow_ref[...] == lax.broadcasted_iota(jnp.int32, (B, NP), 0)
    ohf_t = onehot_t.astype(jnp.float32)

    gneg = jnp.where(onehot, gate, -3.4e38)
    gmax = jnp.max(gneg, axis=0, keepdims=True)
    has = jnp.any(onehot, axis=0, keepdims=True)
    gmax = jnp.where(has, gmax, 0.0)
    gsel = jnp.sum(jnp.where(onehot, gmax, 0.0), axis=1, keepdims=True)

    ex = jnp.exp(gate - gsel)
    den_b = jnp.dot(ohf_t, ex, preferred_element_type=jnp.float32)
    densel = jnp.sum(jnp.where(onehot, den_b.reshape(1, B), 0.0),
                     axis=1, keepdims=True)
    densel = jnp.where(densel > 0, densel, 1.0)
    wgt = ex / densel

    pooled = jnp.dot(ohf_t, wgt * h2, preferred_element_type=jnp.float32)
    z = jnp.maximum(jnp.dot(pooled, w3_ref[...], preferred_element_type=jnp.float32) + b3_ref[...], 0.0)
    out_ref[...] = jnp.dot(z, w4_ref[...], preferred_element_type=jnp.float32) + b4_ref[...]


def kernel(x, edge_index, batch, Wl1, bl1, Wr1, br1, att1, bias1, Wl2, bl2,
           Wr2, br2, att2, bias2, Wg1, bg1, Wg2, bg2, W3, b3, W4, b4):
    f32 = jnp.float32
    xp = jnp.pad(x.astype(f32), ((0, NP - N), (0, 0)))
    loop = jnp.arange(N, dtype=edge_index.dtype)
    src = jnp.concatenate([edge_index[0], loop])
    dst = jnp.concatenate([edge_index[1], loop])
    src3 = jnp.pad(src, (0, EP - ET), constant_values=N).reshape(NW, KCH, CHUNK)
    dst3 = jnp.pad(dst, (0, EP - ET), constant_values=N).reshape(NW, KCH, CHUNK)
    bpad = jnp.pad(batch, (0, NP - N), constant_values=B)
    bcol = bpad.reshape(NP, 1)
    brow = bpad.reshape(1, NP)
    z80 = jnp.zeros((NP, 80), f32)
    z48 = jnp.zeros((NP, 48), f32)

    xl1, xr1 = _dual_matmul(xp, Wl1, bl1, Wr1, br1, 64)
    acc1 = _sc_edge_sweep(2, HID, xl1, xr1, src3, dst3, att1, z80)

    xl2, xr2 = pl.pallas_call(
        _mid_body,
        out_shape=[jax.ShapeDtypeStruct((NP, 32), f32)] * 2,
    )(acc1, bias1.reshape(1, 64), Wl2, bl2.reshape(1, 32), Wr2, br2.reshape(1, 32))

    acc2 = _sc_edge_sweep(1, HID, xl2, xr2, src3, dst3, att2, z48)

    out = pl.pallas_call(
        _final_body,
        out_shape=jax.ShapeDtypeStruct((B, 1), f32),
    )(acc2, bias2.reshape(1, 32), bcol, brow, Wg1, bg1.reshape(1, 32),
      Wg2, bg2.reshape(1, 1), W3, b3.reshape(1, 32), W4, b4.reshape(1, 1))
    return out.reshape(B)


# speedup vs baseline: 1.3228x; 1.3228x over previous
import functools

import jax
import jax.numpy as jnp
from jax import lax
from jax.experimental import pallas as pl
from jax.experimental.pallas import tpu as pltpu
from jax.experimental.pallas import tpu_sc as plsc

N = 10000
E = 320000
B = 64
IN = 128
HID = 32

NP = 10240
NW = 32
CHUNK = 128
ET = E + N
KCH = 2 * (-(-ET // (NW * CHUNK * 2)))
EP = NW * KCH * CHUNK
RPT = NP // 16


def _mm2_body(x_ref, wl_ref, blr, wr_ref, brr, xl_ref, xr_ref):
    xv = x_ref[...]
    xl_ref[...] = jnp.dot(xv, wl_ref[...], preferred_element_type=jnp.float32) + blr[...]
    xr_ref[...] = jnp.dot(xv, wr_ref[...], preferred_element_type=jnp.float32) + brr[...]


def _dual_matmul(xp, Wl, bl, Wr, br, dout):
    return pl.pallas_call(
        _mm2_body,
        out_shape=[jax.ShapeDtypeStruct((NP, dout), jnp.float32)] * 2,
    )(xp, Wl, bl.reshape(1, dout), Wr, br.reshape(1, dout))


def _sc_edge_sweep(H, C, xl, xr, src3, dst3, att2d, zrow):
    DIN = H * C
    WOUT = DIN + 16
    NV = DIN // 16
    VPH = C // 16

    rot = (jnp.arange(16)[None, :] + jnp.arange(C)[:, None]) % C
    attrot = att2d[:, rot].reshape(DIN, 16)

    mesh = plsc.VectorSubcoreMesh(core_axis_name="c", subcore_axis_name="s")

    def body(xl_hbm, xr_hbm, src_hbm, dst_hbm, att_hbm, zero_hbm, out_hbm,
             sidx, didx, didxs, xlr, xrr, wbuf, attv, accs,
             isem, gsem, ssem):
        c = lax.axis_index("c")
        s = lax.axis_index("s")
        w = c * 16 + s

        pltpu.sync_copy(zero_hbm.at[pl.ds(s * RPT, RPT)],
                        accs.at[pl.ds(s * RPT, RPT)])
        pltpu.sync_copy(att_hbm, attv)
        plsc.subcore_barrier()

        io = lax.iota(jnp.int32, 16)
        cmask = jnp.full((16,), C - 1, jnp.int32)
        PF = 10

        def compute_chunk(a):
            def group(g, carry2):
                rows = g * 16 + io

                def cvec_of(h, t):
                    return ((io + t) & cmask) + (h * C)

                def loads_at(h, t):
                    cv = cvec_of(h, t)
                    return (cv,
                            plsc.load_gather(xlr.at[a], [rows, cv]),
                            plsc.load_gather(xrr.at[a], [rows, cv]))

                hts = [(h, t) for h in range(H) for t in range(C)]
                pend = [loads_at(*hts[i]) for i in range(PF)]
                accs_h = [jnp.zeros((16,), jnp.float32) for _ in range(H)]
                for i, (h, t) in enumerate(hts):
                    if i + PF < len(hts):
                        pend.append(loads_at(*hts[i + PF]))
                    _, xa, xb = pend.pop(0)
                    m = xa + xb
                    lk = jnp.maximum(m, 0.2 * m)
                    accs_h[h] = accs_h[h] + lk * attv[h * C + t]
                exv_h = [jnp.exp(acc) for acc in accs_h]

                def oloads_at(h, t):
                    cv = cvec_of(h, t)
                    return (cv, plsc.load_gather(xlr.at[a], [rows, cv]))

                pend = [oloads_at(*hts[i]) for i in range(PF)]
                for i, (h, t) in enumerate(hts):
                    if i + PF < len(hts):
                        pend.append(oloads_at(*hts[i + PF]))
                    cv, xa = pend.pop(0)
                    plsc.store_scatter(wbuf.at[a], [rows, cv], exv_h[h] * xa)
                for h in range(H):
                    dcol = io * 0 + (DIN + h)
                    plsc.store_scatter(wbuf.at[a], [rows, dcol], exv_h[h])
                return carry2

            lax.fori_loop(0, CHUNK // 16, group, 0)

        def wait_gathers(a):
            pltpu.make_async_copy(xl_hbm.at[sidx.at[a]], xlr.at[a], gsem.at[a]).wait()
            pltpu.make_async_copy(xr_hbm.at[didx.at[a]], xrr.at[a], gsem.at[a]).wait()

        def fire_gathers(a):
            pltpu.async_copy(xl_hbm.at[sidx.at[a]], xlr.at[a], gsem.at[a])
            pltpu.async_copy(xr_hbm.at[didx.at[a]], xrr.at[a], gsem.at[a])

        def fire_idx(j, a):
            pltpu.async_copy(src_hbm.at[w, j], sidx.at[a], isem.at[a])
            pltpu.async_copy(dst_hbm.at[w, j], didx.at[a], isem.at[a])

        def wait_idx(a):
            pltpu.make_async_copy(src_hbm.at[w, 0], sidx.at[a], isem.at[a]).wait()
            pltpu.make_async_copy(dst_hbm.at[w, 0], didx.at[a], isem.at[a]).wait()

        def fire_scatter(a):
            pltpu.async_copy(wbuf.at[a], accs.at[didxs.at[a]], ssem.at[a], add=True)

        def wait_scatter(a):
            pltpu.make_async_copy(wbuf.at[a], accs.at[didxs.at[a]], ssem.at[a]).wait()

        pltpu.sync_copy(src_hbm.at[w, 0], sidx.at[0])
        pltpu.sync_copy(dst_hbm.at[w, 0], didx.at[0])
        fire_gathers(0)
        fire_idx(1, 1)

        def halfstep(j, a, b):
            wait_gathers(a)

            @pl.when(j >= 2)
            def _():
                wait_scatter(a)

            for k in range(CHUNK // 16):
                didxs[a, pl.ds(k * 16, 16)] = didx[a, pl.ds(k * 16, 16)]

            @pl.when(j + 1 < KCH)
            def _():
                wait_idx(b)
                fire_gathers(b)

            @pl.when(j + 2 < KCH)
            def _():
                fire_idx(j + 2, a)

            compute_chunk(a)
            fire_scatter(a)

        def pipe(jj, carry):
            halfstep(2 * jj, 0, 1)
            halfstep(2 * jj + 1, 1, 0)
            return carry

        lax.fori_loop(0, KCH // 2, pipe, 0)
        wait_scatter(0)
        wait_scatter(1)
        plsc.subcore_barrier()
        pltpu.sync_copy(accs.at[pl.ds(s * RPT, RPT)],
                        out_hbm.at[c, pl.ds(s * RPT, RPT)])

    f = pl.kernel(
        body,
        out_type=jax.ShapeDtypeStruct((2, NP, WOUT), jnp.float32),
        mesh=mesh,
        compiler_params=pltpu.CompilerParams(
            needs_layout_passes=False, use_tc_tiling_on_sc=False),
        scratch_types=[
            pltpu.VMEM((2, CHUNK), jnp.int32),
            pltpu.VMEM((2, CHUNK), jnp.int32),
            pltpu.VMEM((2, CHUNK), jnp.int32),
            pltpu.VMEM((2, CHUNK, DIN), jnp.float32),
            pltpu.VMEM((2, CHUNK, DIN), jnp.float32),
            pltpu.VMEM((2, CHUNK, WOUT), jnp.float32),
            pltpu.VMEM((DIN, 16), jnp.float32),
            pltpu.VMEM_SHARED((NP, WOUT), jnp.float32),
            pltpu.SemaphoreType.DMA((2,)),
            pltpu.SemaphoreType.DMA((2,)),
            pltpu.SemaphoreType.DMA((2,)),
        ],
    )
    return f(xl, xr, src3, dst3, attrot, zrow)


def _mid_body(acc_ref, bias_ref, wl_ref, blr, wr_ref, brr, xl_ref, xr_ref):
    a = acc_ref[0] + acc_ref[1]
    num = a[:, :64]
    den = jnp.concatenate(
        [jnp.broadcast_to(a[:, 64:65], (NP, 32)),
         jnp.broadcast_to(a[:, 65:66], (NP, 32))], axis=1)
    h = jnp.maximum(num / den + bias_ref[...], 0.0)
    xl_ref[...] = jnp.dot(h, wl_ref[...], preferred_element_type=jnp.float32) + blr[...]
    xr_ref[...] = jnp.dot(h, wr_ref[...], preferred_element_type=jnp.float32) + brr[...]


def _final_body(acc_ref, bias_ref, bcol_ref, brow_ref, wg1_ref, bg1_ref,
                wg2_ref, bg2_ref, w3_ref, b3_ref, w4_ref, b4_ref, out_ref):
    a = acc_ref[0] + acc_ref[1]
    h2 = jnp.maximum(a[:, :32] / a[:, 32:33] + bias_ref[...], 0.0)
    rows = lax.broadcasted_iota(jnp.int32, (NP, 1), 0)
    h2 = jnp.where(rows < N, h2, 0.0)

    g1 = jnp.maximum(jnp.dot(h2, wg1_ref[...], preferred_element_type=jnp.float32) + bg1_ref[...], 0.0)
    gate = jnp.dot(g1, wg2_ref[...], preferred_element_type=jnp.float32) + bg2_ref[...]

    bcol = bcol_ref[...]
    onehot = bcol == lax.broadcasted_iota(jnp.int32, (NP, B), 1)
    onehot_t = brow_ref[...] == lax.broadcasted_iota(jnp.int32, (B, NP), 0)
    ohf_t = onehot_t.astype(jnp.float32)

    gneg = jnp.where(onehot, gate, -3.4e38)
    gmax = jnp.max(gneg, axis=0, keepdims=True)
    has = jnp.any(onehot, axis=0, keepdims=True)
    gmax = jnp.where(has, gmax, 0.0)
    gsel = jnp.sum(jnp.where(onehot, gmax, 0.0), axis=1, keepdims=True)

    ex = jnp.exp(gate - gsel)
    den_b = jnp.dot(ohf_t, ex, preferred_element_type=jnp.float32)
    densel = jnp.sum(jnp.where(onehot, den_b.reshape(1, B), 0.0),
                     axis=1, keepdims=True)
    densel = jnp.where(densel > 0, densel, 1.0)
    wgt = ex / densel

    pooled = jnp.dot(ohf_t, wgt * h2, preferred_element_type=jnp.float32)
    z = jnp.maximum(jnp.dot(pooled, w3_ref[...], preferred_element_type=jnp.float32) + b3_ref[...], 0.0)
    out_ref[...] = jnp.dot(z, w4_ref[...], preferred_element_type=jnp.float32) + b4_ref[...]


def kernel(x, edge_index, batch, Wl1, bl1, Wr1, br1, att1, bias1, Wl2, bl2,
           Wr2, br2, att2, bias2, Wg1, bg1, Wg2, bg2, W3, b3, W4, b4):
    f32 = jnp.float32
    xp = jnp.pad(x.astype(f32), ((0, NP - N), (0, 0)))
    loop = jnp.arange(N, dtype=edge_index.dtype)
    src = jnp.concatenate([edge_index[0], loop])
    dst = jnp.concatenate([edge_index[1], loop])
    src3 = jnp.pad(src, (0, EP - ET), constant_values=N).reshape(NW, KCH, CHUNK)
    dst3 = jnp.pad(dst, (0, EP - ET), constant_values=N).reshape(NW, KCH, CHUNK)
    bpad = jnp.pad(batch, (0, NP - N), constant_values=B)
    bcol = bpad.reshape(NP, 1)
    brow = bpad.reshape(1, NP)
    z80 = jnp.zeros((NP, 80), f32)
    z48 = jnp.zeros((NP, 48), f32)

    xl1, xr1 = _dual_matmul(xp, Wl1, bl1, Wr1, br1, 64)
    acc1 = _sc_edge_sweep(2, HID, xl1, xr1, src3, dst3, att1, z80)

    xl2, xr2 = pl.pallas_call(
        _mid_body,
        out_shape=[jax.ShapeDtypeStruct((NP, 32), f32)] * 2,
    )(acc1, bias1.reshape(1, 64), Wl2, bl2.reshape(1, 32), Wr2, br2.reshape(1, 32))

    acc2 = _sc_edge_sweep(1, HID, xl2, xr2, src3, dst3, att2, z48)

    out = pl.pallas_call(
        _final_body,
        out_shape=jax.ShapeDtypeStruct((B, 1), f32),
    )(acc2, bias2.reshape(1, 32), bcol, brow, Wg1, bg1.reshape(1, 32),
      Wg2, bg2.reshape(1, 1), W3, b3.reshape(1, 32), W4, b4.reshape(1, 1))
    return out.reshape(B)
